# baseline (speedup 1.0000x reference)
import jax
import jax.numpy as jnp
from jax.experimental import pallas as pl
from jax.experimental.pallas import tpu as pltpu

B = 16
N = 4096
D = 256
BLK = 512
NBLK = N // BLK


def _graph_kernel(starts_ref, ends_ref, x_ref, wb_ref, bb_ref, wp_ref,
                  bp_ref, wr_ref, br_ref, ap_ref, wqp_ref, ar_ref, wqr_ref,
                  keys_ref, pq_ref, rq_ref,
                  kacc_ref, macc_ref, lacc_ref, pacc_ref):
    pid = pl.program_id(0)
    bf16 = jnp.bfloat16

    @pl.when(pid == 0)
    def _init():
        kacc_ref[...] = jnp.zeros_like(kacc_ref)
        macc_ref[...] = jnp.full_like(macc_ref, -1e30)
        lacc_ref[...] = jnp.zeros_like(lacc_ref)
        pacc_ref[...] = jnp.zeros_like(pacc_ref)

    x = x_ref[...].astype(bf16)
    feats = jnp.maximum(jnp.dot(x, wb_ref[...],
                                preferred_element_type=jnp.float32)
                        + bb_ref[...], 0.0)
    featsb = feats.astype(bf16)

    ids = jax.lax.broadcasted_iota(jnp.int32, (B, BLK), 1) + pid * BLK
    starts = starts_ref[...]
    ends = ends_ref[...]
    seg = jnp.logical_and(ids >= starts, ids < ends)

    kacc_ref[...] += jnp.dot(seg.astype(bf16), featsb,
                             preferred_element_type=jnp.float32)

    def branch(idx, w_ref, b_ref, att_ref):
        feat = jnp.maximum(jnp.dot(featsb, w_ref[...],
                                   preferred_element_type=jnp.float32)
                           + b_ref[...], 0.0)
        featb = feat.astype(bf16)
        scores = jax.lax.dot_general(
            att_ref[...], feat, (((1,), (1,)), ((), ())),
            preferred_element_type=jnp.float32)
        s2 = jnp.where(seg, scores, -jnp.inf)
        blk_m = jnp.max(s2, axis=1, keepdims=True)
        m_old = macc_ref[idx]
        m_new = jnp.maximum(m_old, blk_m)
        scale = jnp.exp(m_old - m_new)
        e = jnp.exp(s2 - m_new)
        macc_ref[idx] = m_new
        lacc_ref[idx] = lacc_ref[idx] * scale + jnp.sum(e, axis=1,
                                                        keepdims=True)
        pacc_ref[idx] = pacc_ref[idx] * scale + jnp.dot(
            e.astype(bf16), featb, preferred_element_type=jnp.float32)

    branch(0, wp_ref, bp_ref, ap_ref)
    branch(1, wr_ref, br_ref, ar_ref)

    @pl.when(pid == NBLK - 1)
    def _finalize():
        inv_len = 1.0 / jnp.maximum((ends - starts).astype(jnp.float32),
                                    1.0)
        keys_ref[...] = kacc_ref[...] * inv_len
        inv_l = 1.0 / jnp.maximum(lacc_ref[...], 1e-30)
        pooled = pacc_ref[...] * inv_l
        pq_ref[...] = jnp.dot(pooled[0], wqp_ref[...],
                              preferred_element_type=jnp.float32)
        rq_ref[...] = jnp.dot(pooled[1], wqr_ref[...],
                              preferred_element_type=jnp.float32)


def kernel(x, cu_seqlens, W_base, b_base, W_p, b_p, W_r, b_r,
           w_att_p, W_q_p, w_att_r, W_q_r):
    cu = cu_seqlens.astype(jnp.int32)
    starts = cu[:-1].reshape(B, 1)
    ends = cu[1:].reshape(B, 1)
    full = lambda shape: pl.BlockSpec(shape, lambda i: (0,) * len(shape))
    in_specs = [
        full((B, 1)),
        full((B, 1)),
        pl.BlockSpec((BLK, D), lambda i: (i, 0)),
        full((D, D)), full((1, D)),
        full((D, D)), full((1, D)),
        full((D, D)), full((1, D)),
        full((1, D)), full((D, D)),
        full((1, D)), full((D, D)),
    ]
    out_specs = (full((B, D)), full((B, D)), full((B, D)))
    out_shape = tuple(jax.ShapeDtypeStruct((B, D), jnp.float32)
                      for _ in range(3))
    bf16 = jnp.bfloat16
    return pl.pallas_call(
        _graph_kernel,
        grid=(NBLK,),
        in_specs=in_specs,
        out_specs=out_specs,
        out_shape=out_shape,
        scratch_shapes=[
            pltpu.VMEM((B, D), jnp.float32),
            pltpu.VMEM((2, B, 1), jnp.float32),
            pltpu.VMEM((2, B, 1), jnp.float32),
            pltpu.VMEM((2, B, D), jnp.float32),
        ],
    )(starts, ends, x,
      W_base.astype(bf16), b_base.reshape(1, D),
      W_p.astype(bf16), b_p.reshape(1, D),
      W_r.astype(bf16), b_r.reshape(1, D),
      w_att_p.reshape(1, D), W_q_p,
      w_att_r.reshape(1, D), W_q_r)


# speedup vs baseline: 1.1918x; 1.1918x over previous
import jax
import jax.numpy as jnp
from jax.experimental import pallas as pl
from jax.experimental.pallas import tpu as pltpu

B = 16
N = 4096
D = 256
CHUNK = 1024
NCHUNK = N // CHUNK


def _graph_kernel(x_hbm, starts_ref, ends_ref, wb_ref, bb_ref, wp_ref,
                  bp_ref, wr_ref, br_ref, ap_ref, wqp_ref, ar_ref, wqr_ref,
                  keys_ref, pq_ref, rq_ref,
                  xbuf, featsb_ref, sems):
    bf16 = jnp.bfloat16

    def chunk_copy(i, slot):
        return pltpu.make_async_copy(
            x_hbm.at[pl.ds(i * CHUNK, CHUNK), :], xbuf.at[slot],
            sems.at[slot])

    chunk_copy(0, 0).start()
    for i in range(NCHUNK):
        if i + 1 < NCHUNK:
            chunk_copy(i + 1, (i + 1) % 2).start()
        chunk_copy(i, i % 2).wait()
        xc = xbuf[i % 2].astype(bf16)
        fc = jnp.maximum(jnp.dot(xc, wb_ref[...],
                                 preferred_element_type=jnp.float32)
                         + bb_ref[...], 0.0)
        featsb_ref[pl.ds(i * CHUNK, CHUNK), :] = fc.astype(bf16)

    featsb = featsb_ref[...]
    ids = jax.lax.broadcasted_iota(jnp.int32, (B, N), 1)
    starts = starts_ref[...]
    ends = ends_ref[...]
    seg = jnp.logical_and(ids >= starts, ids < ends)

    seg_sum = jnp.dot(seg.astype(bf16), featsb,
                      preferred_element_type=jnp.float32)
    inv_len = 1.0 / jnp.maximum((ends - starts).astype(jnp.float32), 1.0)
    keys_ref[...] = seg_sum * inv_len

    def branch(w_ref, b_ref, att_ref, wq_ref, out_ref):
        feat = jnp.maximum(jnp.dot(featsb, w_ref[...],
                                   preferred_element_type=jnp.float32)
                           + b_ref[...], 0.0)
        featb = feat.astype(bf16)
        scores = jax.lax.dot_general(
            att_ref[...], feat, (((1,), (1,)), ((), ())),
            preferred_element_type=jnp.float32)
        s2 = jnp.where(seg, scores, -jnp.inf)
        m = jnp.max(s2, axis=1, keepdims=True)
        e = jnp.exp(s2 - m)
        l = jnp.sum(e, axis=1, keepdims=True)
        attn = (e * (1.0 / jnp.maximum(l, 1e-30))).astype(bf16)
        pooled = jnp.dot(attn, featb, preferred_element_type=jnp.float32)
        out_ref[...] = jnp.dot(pooled, wq_ref[...],
                               preferred_element_type=jnp.float32)

    branch(wp_ref, bp_ref, ap_ref, wqp_ref, pq_ref)
    branch(wr_ref, br_ref, ar_ref, wqr_ref, rq_ref)


def kernel(x, cu_seqlens, W_base, b_base, W_p, b_p, W_r, b_r,
           w_att_p, W_q_p, w_att_r, W_q_r):
    cu = cu_seqlens.astype(jnp.int32)
    starts = cu[:-1].reshape(B, 1)
    ends = cu[1:].reshape(B, 1)
    bf16 = jnp.bfloat16
    in_specs = ([pl.BlockSpec(memory_space=pltpu.MemorySpace.HBM)] +
                [pl.BlockSpec(memory_space=pltpu.MemorySpace.VMEM)
                 for _ in range(12)])
    out_shape = tuple(jax.ShapeDtypeStruct((B, D), jnp.float32)
                      for _ in range(3))
    return pl.pallas_call(
        _graph_kernel,
        in_specs=in_specs,
        out_shape=out_shape,
        scratch_shapes=[
            pltpu.VMEM((2, CHUNK, D), jnp.float32),
            pltpu.VMEM((N, D), bf16),
            pltpu.SemaphoreType.DMA((2,)),
        ],
    )(x, starts, ends,
      W_base.astype(bf16), b_base.reshape(1, D),
      W_p.astype(bf16), b_p.reshape(1, D),
      W_r.astype(bf16), b_r.reshape(1, D),
      w_att_p.reshape(1, D), W_q_p,
      w_att_r.reshape(1, D), W_q_r)
